# baseline (speedup 1.0000x reference)
import functools

import jax
import jax.numpy as jnp
from jax import lax
from jax.experimental import pallas as pl
from jax.experimental.pallas import tpu as pltpu
from jax.experimental.pallas import tpu_sc as plsc

N_FACTORS = 32
BATCH = 16384
NC, NS, L = 2, 16, 16
NW = NC * NS
BPW = BATCH // NW
CHUNK = 128
NCHUNK = BPW // CHUNK
GROUPS = BPW // L


def _tree_sum(vs):
    while len(vs) > 1:
        vs = [a + b for a, b in zip(vs[::2], vs[1::2])]
    return vs[0]


def _gmfb_body(user_hbm, item_hbm, wb_hbm, u_tab, i_tab, out_hbm,
               idx_u, idx_i, u_rows, i_rows, t_v, out_v, w_v, sem):
    cid = lax.axis_index("c")
    sid = lax.axis_index("s")
    wid = sid * NC + cid
    base = wid * BPW

    pltpu.sync_copy(user_hbm.at[wid], idx_u)
    pltpu.sync_copy(item_hbm.at[wid], idx_i)
    pltpu.sync_copy(wb_hbm, w_v)

    copies = []
    for j in range(NCHUNK):
        copies.append(pltpu.make_async_copy(
            u_tab.at[idx_u.at[j]], u_rows.at[pl.ds(j * CHUNK, CHUNK)], sem))
        copies.append(pltpu.make_async_copy(
            i_tab.at[idx_i.at[j]], i_rows.at[pl.ds(j * CHUNK, CHUNK)], sem))
    for cp in copies:
        cp.start()
    for cp in copies:
        cp.wait()

    w0a = w_v[pl.ds(0, L)]
    w0b = w_v[pl.ds(16, L)]
    w1a = w_v[pl.ds(32, L)]
    w1b = w_v[pl.ds(48, L)]
    w2a = w_v[pl.ds(64, L)]
    w2b = w_v[pl.ds(80, L)]
    bvec = w_v[pl.ds(96, L)]

    lane = lax.iota(jnp.int32, L)

    def row_body(r, carry):
        u1 = u_rows[r, pl.ds(0, L)]
        u2 = u_rows[r, pl.ds(16, L)]
        i1 = i_rows[r, pl.ds(0, L)]
        i2 = i_rows[r, pl.ds(16, L)]
        t = (u1 * i1 * w0a + u1 * w1a + i1 * w2a
             + u2 * i2 * w0b + u2 * w1b + i2 * w2b)
        plsc.store_scatter(t_v, [lane * BPW + r], t)
        return carry

    lax.fori_loop(0, BPW, row_body, 0)

    def grp_body(g, carry):
        vals = [t_v[pl.ds(k * BPW + g * L, L)] for k in range(L)]
        x = _tree_sum(vals) + bvec
        y = 1.0 / (1.0 + jnp.exp(-x))
        out_v[pl.ds(g * L, L)] = y
        return carry

    lax.fori_loop(0, GROUPS, grp_body, 0)

    pltpu.sync_copy(out_v, out_hbm.at[pl.ds(base, BPW)])


@functools.partial(jax.jit, static_argnames=())
def _gmfb(user_r, item_r, wb, user_emb, item_emb):
    mesh = plsc.VectorSubcoreMesh(core_axis_name="c", subcore_axis_name="s",
                                  num_cores=NC, num_subcores=NS)
    f = pl.kernel(
        _gmfb_body,
        out_type=jax.ShapeDtypeStruct((BATCH,), jnp.float32),
        mesh=mesh,
        scratch_types=[
            pltpu.VMEM((NCHUNK, CHUNK), jnp.int32),
            pltpu.VMEM((NCHUNK, CHUNK), jnp.int32),
            pltpu.VMEM((BPW, N_FACTORS), jnp.float32),
            pltpu.VMEM((BPW, N_FACTORS), jnp.float32),
            pltpu.VMEM((L * BPW,), jnp.float32),
            pltpu.VMEM((BPW,), jnp.float32),
            pltpu.VMEM((112,), jnp.float32),
            pltpu.SemaphoreType.DMA,
        ],
        compiler_params=pltpu.CompilerParams(needs_layout_passes=False,
                                             use_tc_tiling_on_sc=False),
    )
    return f(user_r, item_r, wb, user_emb, item_emb)


def kernel(user, item, user_emb, item_emb, h_w, h_b):
    user_r = user.reshape(NW, NCHUNK, CHUNK)
    item_r = item.reshape(NW, NCHUNK, CHUNK)
    wb = jnp.concatenate([h_w.reshape(N_FACTORS * 3),
                          jnp.broadcast_to(h_b.reshape(1), (L,))])
    return _gmfb(user_r, item_r, wb, user_emb, item_emb)
